# baseline (speedup 1.0000x reference)
import functools

import jax
import jax.numpy as jnp
from jax import lax
from jax.experimental import pallas as pl
from jax.experimental.pallas import tpu as pltpu
from jax.experimental.pallas import tpu_sc as plsc

N = 10000
E = 320000
DIN = 128
DOUT = 128

NC = 2
NS = 16
NW = NC * NS
EP = E // NW
K = 80
NCH = EP // K
NPAD = 10240
ROWS_PER_TILE = NPAD // NS
FB = DOUT // 16


def _mm_body(x_ref, w_ref, h_ref):
    h_ref[...] = lax.dot_general(
        x_ref[...], w_ref[...], (((1,), (1,)), ((), ())),
        preferred_element_type=jnp.float32)


def _add_body(a_ref, b_ref, o_ref):
    o_ref[...] = a_ref[...] + b_ref[...]


def _sc_body(h_hbm, comb_hbm, out_hbm,
             idx0, idx1, rows0, rows1, sout_v, acc_sh,
             si0, si1, sg0, sg1, ssc):
    c = lax.axis_index("c")
    s = lax.axis_index("s")
    wid = s * NC + c

    idxs = (idx0, idx1)
    rowss = (rows0, rows1)
    sis = (si0, si1)
    sgs = (sg0, sg1)

    def fetch_idx(j, m):
        pltpu.async_copy(comb_hbm.at[wid, j], idxs[m], sis[m])

    def wait_idx(m):
        pltpu.make_async_copy(comb_hbm.at[wid, 0], idxs[m], sis[m]).wait()

    def fire_gather(m):
        pltpu.async_copy(h_hbm.at[idxs[m].at[0, 0]], rowss[m], sgs[m])

    def wait_gather(m):
        pltpu.make_async_copy(h_hbm.at[idxs[m].at[0, 0]], rowss[m],
                              sgs[m]).wait()

    def scale(m):
        rows_v = rowss[m]
        idx_v = idxs[m]

        def grp_body(g, gcarry):
            w16 = lax.bitcast_convert_type(idx_v[2, 0, pl.ds(g * 16, 16)],
                                           jnp.float32)
            base = g * 16
            for e in range(16):
                w = lax.gather(
                    w16, jnp.full((16, 1), e, jnp.int32),
                    lax.GatherDimensionNumbers(
                        offset_dims=(), collapsed_slice_dims=(0,),
                        start_index_map=(0,)),
                    (1,), mode=lax.GatherScatterMode.PROMISE_IN_BOUNDS)
                for f in range(FB):
                    sl = pl.ds(16 * f, 16)
                    sout_v[base + e, sl] = rows_v[base + e, sl] * w
            return gcarry

        lax.fori_loop(0, K // 16, grp_body, 0)

    def scatter(m):
        pltpu.async_copy(sout_v, acc_sh.at[idxs[m].at[1, 0]], ssc,
                         add=True).wait()

    fetch_idx(0, 0)
    fetch_idx(1, 1)

    zero16 = jnp.zeros((16,), jnp.float32)

    def zrow(i, carry):
        for f in range(FB):
            sout_v[i, pl.ds(16 * f, 16)] = zero16
        return carry

    lax.fori_loop(0, K, zrow, 0)
    for r in range(ROWS_PER_TILE // K):
        pltpu.sync_copy(sout_v,
                        acc_sh.at[pl.ds(s * ROWS_PER_TILE + r * K, K)])

    wait_idx(0)
    fire_gather(0)
    plsc.subcore_barrier()

    def chunk_step(j, m, fire_next, fetch_mode):
        wait_gather(m)
        scale(m)
        if fire_next:
            wait_idx(1 - m)
            fire_gather(1 - m)
        scatter(m)
        if fetch_mode == "always":
            fetch_idx(j + 2, m)
        elif fetch_mode == "guard":
            @pl.when(j + 2 < NCH)
            def _():
                fetch_idx(j + 2, m)

    def pair_body(p, carry):
        j0 = 2 * p
        chunk_step(j0, 0, True, "always")
        chunk_step(j0 + 1, 1, True, "guard")
        return carry

    lax.fori_loop(0, (NCH - 1) // 2, pair_body, 0)
    chunk_step(NCH - 1, 0, False, "none")

    plsc.subcore_barrier()

    pltpu.sync_copy(acc_sh.at[pl.ds(s * ROWS_PER_TILE, ROWS_PER_TILE)],
                    out_hbm.at[pl.ds(c * NPAD + s * ROWS_PER_TILE,
                                     ROWS_PER_TILE)])


@functools.cache
def _sc_gather_scale_scatter():
    return pl.kernel(
        _sc_body,
        out_type=jax.ShapeDtypeStruct((NC * NPAD, DOUT), jnp.float32),
        mesh=plsc.VectorSubcoreMesh(core_axis_name="c", subcore_axis_name="s",
                                    num_cores=NC, num_subcores=NS),
        scratch_types=[
            pltpu.VMEM((3, 1, K), jnp.int32),
            pltpu.VMEM((3, 1, K), jnp.int32),
            pltpu.VMEM((K, DOUT), jnp.float32),
            pltpu.VMEM((K, DOUT), jnp.float32),
            pltpu.VMEM((K, DOUT), jnp.float32),
            pltpu.VMEM_SHARED((NPAD, DOUT), jnp.float32),
            pltpu.SemaphoreType.DMA,
            pltpu.SemaphoreType.DMA,
            pltpu.SemaphoreType.DMA,
            pltpu.SemaphoreType.DMA,
            pltpu.SemaphoreType.DMA,
        ],
    )


@jax.jit
def kernel(x, edge_index, edge_weight, W):
    h = pl.pallas_call(
        _mm_body,
        grid=(10,),
        in_specs=[
            pl.BlockSpec((N // 10, DIN), lambda i: (i, 0)),
            pl.BlockSpec((DOUT, DIN), lambda i: (0, 0)),
        ],
        out_specs=pl.BlockSpec((N // 10, DOUT), lambda i: (i, 0)),
        out_shape=jax.ShapeDtypeStruct((N, DOUT), jnp.float32),
    )(x, W)

    dst = edge_index[0].reshape(NW, NCH, 1, 1, K)
    src = edge_index[1].reshape(NW, NCH, 1, 1, K)
    ewb = lax.bitcast_convert_type(edge_weight, jnp.int32).reshape(
        NW, NCH, 1, 1, K)
    comb = jnp.concatenate([src, dst, ewb], axis=2)

    partial = _sc_gather_scale_scatter()(h, comb)

    spec = pl.BlockSpec((N // 10, DOUT), lambda i: (i, 0))
    out = pl.pallas_call(
        _add_body,
        grid=(10,),
        in_specs=[spec, spec],
        out_specs=spec,
        out_shape=jax.ShapeDtypeStruct((N, DOUT), jnp.float32),
    )(partial[:N], partial[NPAD:NPAD + N])
    return out
